# baseline (speedup 1.0000x reference)
import functools

import jax
import jax.numpy as jnp
from jax import lax
from jax.experimental import pallas as pl
from jax.experimental.pallas import tpu as pltpu
from jax.experimental.pallas import tpu_sc as plsc

B = 16
L = 2048
D = 1024
N = 16384

NC = 2
NS = 16
NW = NC * NS
RPW = (B * L) // NW
GC = 32
ZB = 128
NCHUNK = RPW // GC


def _sc_body(src_hbm, batch_hbm, zeros_hbm, out_hbm, mask_hbm,
             batch_v, bufA, bufB, zshared, idx2, maskbuf,
             sem_gA, sem_gB, sem_wA, sem_wB, sem_z):
    wid = lax.axis_index("s") * NC + lax.axis_index("c")
    b = wid & (B - 1)
    p0 = (wid >> 4) * RPW

    pltpu.sync_copy(batch_hbm, batch_v)

    @pl.when(lax.axis_index("s") == 0)
    def _zinit():
        pltpu.sync_copy(zeros_hbm, zshared)

    plsc.subcore_barrier()

    z16 = jnp.zeros((16,), jnp.int32)
    one16 = jnp.full((16,), 1, jnp.int32)
    bvec = jnp.full((16,), b, jnp.int32)

    def scan_body(i, carry):
        lt, le = carry
        v = batch_v[pl.ds(i * 16, 16)]
        lt = lt + jnp.where(v < bvec, one16, z16)
        le = le + jnp.where(v <= bvec, one16, z16)
        return lt, le

    lt, le = lax.fori_loop(0, N // 16, scan_body, (z16, z16))
    start_b = jnp.sum(lt)
    end_b = jnp.sum(le)
    count_b = end_b - start_b

    valid = jnp.clip(count_b - p0, 0, RPW)
    src0 = start_b + p0
    obase = b * L + p0

    nfull = valid // GC
    rem = valid - nfull * GC
    jz0 = nfull + jnp.where(rem > 0, 1, 0)

    iota16 = lax.iota(jnp.int32, 16)

    pmax = jnp.maximum(valid - 1, 0)

    cb = jnp.full((16,), count_b, jnp.int32)
    pfull0 = jnp.full((16,), p0, jnp.int32)

    def idx_body(k, _):
        j = k // (GC // 16)
        col = (k % (GC // 16)) * 16
        p = jnp.full((16,), k * 16, jnp.int32) + iota16
        pc = jnp.minimum(p, jnp.full((16,), pmax, jnp.int32))
        idx2[j, pl.ds(col, 16)] = pc + jnp.full((16,), src0, jnp.int32)
        maskbuf[pl.ds(k * 16, 16)] = jnp.where(p + pfull0 < cb, one16, z16)
        return 0

    lax.fori_loop(0, RPW // 16, idx_body, 0)

    z0 = jz0 * GC
    nz = RPW - z0
    n128 = nz // ZB
    n32 = (nz - n128 * ZB) // GC

    def zbig_start(k, _):
        pltpu.make_async_copy(zshared,
                              out_hbm.at[pl.ds(obase + RPW - (k + 1) * ZB, ZB)],
                              sem_z).start()
        return 0

    lax.fori_loop(0, n128, zbig_start, 0)

    def zsmall_start(k, _):
        pltpu.make_async_copy(zshared.at[pl.ds(0, GC)],
                              out_hbm.at[pl.ds(obase + z0 + k * GC, GC)],
                              sem_z).start()
        return 0

    lax.fori_loop(0, n32, zsmall_start, 0)

    def gather_start(j, buf, sem):
        pltpu.make_async_copy(src_hbm.at[idx2.at[j]], buf, sem).start()

    def gather_wait(buf, sem):
        pltpu.make_async_copy(src_hbm.at[idx2.at[0]], buf, sem).wait()

    def write_start(j, buf, sem):
        pltpu.make_async_copy(buf, out_hbm.at[pl.ds(obase + j * GC, GC)],
                              sem).start()

    def write_wait(buf, sem):
        pltpu.make_async_copy(buf, out_hbm.at[pl.ds(obase, GC)], sem).wait()

    def pipe_iter(j, cur, sem_gc, sem_wc, prv, sem_gp, sem_wp):
        @pl.when(j >= 2)
        def _():
            write_wait(cur, sem_wc)
        gather_start(j, cur, sem_gc)
        @pl.when(j >= 1)
        def _():
            gather_wait(prv, sem_gp)
            write_start(j - 1, prv, sem_wp)

    def pipe_body(j, _):
        @pl.when((j & 1) == 0)
        def _():
            pipe_iter(j, bufA, sem_gA, sem_wA, bufB, sem_gB, sem_wB)
        @pl.when((j & 1) == 1)
        def _():
            pipe_iter(j, bufB, sem_gB, sem_wB, bufA, sem_gA, sem_wA)
        return 0

    lax.fori_loop(0, nfull, pipe_body, 0)

    @pl.when(nfull >= 1)
    def _epilogue():
        jl = nfull - 1

        @pl.when((jl & 1) == 0)
        def _():
            gather_wait(bufA, sem_gA)
            write_start(jl, bufA, sem_wA)
            write_wait(bufA, sem_wA)
        @pl.when((jl & 1) == 1)
        def _():
            gather_wait(bufB, sem_gB)
            write_start(jl, bufB, sem_wB)
            write_wait(bufB, sem_wB)

        @pl.when(nfull >= 2)
        def _():
            @pl.when((jl & 1) == 0)
            def _():
                write_wait(bufB, sem_wB)
            @pl.when((jl & 1) == 1)
            def _():
                write_wait(bufA, sem_wA)

    @pl.when(rem > 0)
    def _boundary():
        pltpu.async_copy(src_hbm.at[idx2.at[nfull]], bufA, sem_gA).wait()

        def zrow(r, _):
            def zcol(k, _):
                bufA[r, pl.ds(k * 16, 16)] = jnp.zeros((16,), jnp.float32)
                return 0
            lax.fori_loop(0, D // 16, zcol, 0)
            return 0

        lax.fori_loop(rem, GC, zrow, 0)
        pltpu.sync_copy(bufA, out_hbm.at[pl.ds(obase + nfull * GC, GC)])

    pltpu.sync_copy(maskbuf, mask_hbm.at[pl.ds(obase, RPW)])

    def zbig_drain(k, _):
        pltpu.make_async_copy(zshared, out_hbm.at[pl.ds(obase, ZB)],
                              sem_z).wait()
        return 0

    lax.fori_loop(0, n128, zbig_drain, 0)

    def zsmall_drain(k, _):
        pltpu.make_async_copy(zshared.at[pl.ds(0, GC)],
                              out_hbm.at[pl.ds(obase, GC)], sem_z).wait()
        return 0

    lax.fori_loop(0, n32, zsmall_drain, 0)


@functools.partial(
    pl.kernel,
    out_type=(jax.ShapeDtypeStruct((B * L, D), jnp.float32),
              jax.ShapeDtypeStruct((B * L,), jnp.int32)),
    mesh=plsc.VectorSubcoreMesh(core_axis_name="c", subcore_axis_name="s",
                                num_cores=NC, num_subcores=NS),
    scratch_types=[
        pltpu.VMEM((N,), jnp.int32),
        pltpu.VMEM((GC, D), jnp.float32),
        pltpu.VMEM((GC, D), jnp.float32),
        pltpu.VMEM_SHARED((ZB, D), jnp.float32),
        pltpu.VMEM((NCHUNK, GC), jnp.int32),
        pltpu.VMEM((RPW,), jnp.int32),
        pltpu.SemaphoreType.DMA,
        pltpu.SemaphoreType.DMA,
        pltpu.SemaphoreType.DMA,
        pltpu.SemaphoreType.DMA,
        pltpu.SemaphoreType.DMA,
    ],
    compiler_params=pltpu.CompilerParams(needs_layout_passes=False),
)
def _sc_kernel(src_hbm, batch_hbm, zeros_hbm, out_hbm, mask_hbm,
               batch_v, bufA, bufB, zshared, idx2, maskbuf,
               sem_gA, sem_gB, sem_wA, sem_wB, sem_z):
    _sc_body(src_hbm, batch_hbm, zeros_hbm, out_hbm, mask_hbm,
             batch_v, bufA, bufB, zshared, idx2, maskbuf,
             sem_gA, sem_gB, sem_wA, sem_wB, sem_z)


@jax.jit
def kernel(src, batch):
    zeros = jnp.zeros((ZB, D), jnp.float32)
    padded_flat, mask_flat = _sc_kernel(src, batch.astype(jnp.int32), zeros)
    return padded_flat.reshape(B, L, D), mask_flat.reshape(B, L) != 0


# speedup vs baseline: 1.0411x; 1.0411x over previous
import functools

import jax
import jax.numpy as jnp
from jax import lax
from jax.experimental import pallas as pl
from jax.experimental.pallas import tpu as pltpu
from jax.experimental.pallas import tpu_sc as plsc

B = 16
L = 2048
D = 1024
N = 16384

NC = 2
NS = 16
NW = NC * NS
RPW = (B * L) // NW
GC = 32
NCHUNK = RPW // GC


def _sc_body(src_hbm, batch_hbm, zeros_hbm, out_hbm, mask_hbm,
             batch_v, bufA, bufB, zbuf, idx2, maskbuf,
             sem_gA, sem_gB, sem_wA, sem_wB, sem_z):
    wid = lax.axis_index("s") * NC + lax.axis_index("c")
    b = wid & (B - 1)
    q = wid >> 4
    p0 = q * RPW

    pltpu.sync_copy(batch_hbm, batch_v)
    pltpu.sync_copy(zeros_hbm, zbuf)

    z16 = jnp.zeros((16,), jnp.int32)
    one16 = jnp.full((16,), 1, jnp.int32)
    bvec = jnp.full((16,), b, jnp.int32)

    def scan_body(i, carry):
        lt, le = carry
        v = batch_v[pl.ds(i * 16, 16)]
        lt = lt + jnp.where(v < bvec, one16, z16)
        le = le + jnp.where(v <= bvec, one16, z16)
        return lt, le

    lt, le = lax.fori_loop(0, N // 16, scan_body, (z16, z16))
    start_b = jnp.sum(lt)
    end_b = jnp.sum(le)
    count_b = end_b - start_b

    validb = jnp.minimum(count_b, L)
    src0 = start_b
    obase = b * L

    nfullg = validb // GC
    remg = validb - nfullg * GC

    nfull = (nfullg - q + 1) // 2
    has_bnd = jnp.where((remg > 0) & ((nfullg & 1) == q), 1, 0)
    jbnd = (nfullg - q) // 2

    iota16 = lax.iota(jnp.int32, 16)

    pmax = jnp.maximum(validb - 1, 0)
    qGC = q * GC

    def idx_body(k, _):
        j = k // (GC // 16)
        col = (k % (GC // 16)) * 16
        p = (jnp.full((16,), j * 2 * GC + qGC + col, jnp.int32) + iota16)
        pc = jnp.minimum(p, jnp.full((16,), pmax, jnp.int32))
        idx2[j, pl.ds(col, 16)] = pc + jnp.full((16,), src0, jnp.int32)
        return 0

    lax.fori_loop(0, RPW // 16, idx_body, 0)

    def ochunk(j):
        return obase + (2 * j + q) * GC

    jz0 = nfull + has_bnd

    def zero_start(j, _):
        pltpu.make_async_copy(zbuf, out_hbm.at[pl.ds(ochunk(j), GC)],
                              sem_z).start()
        return 0

    lax.fori_loop(jz0, NCHUNK, zero_start, 0)

    def gather_start(j, buf, sem):
        pltpu.make_async_copy(src_hbm.at[idx2.at[j]], buf, sem).start()

    def gather_wait(buf, sem):
        pltpu.make_async_copy(src_hbm.at[idx2.at[0]], buf, sem).wait()

    def write_start(j, buf, sem):
        pltpu.make_async_copy(buf, out_hbm.at[pl.ds(ochunk(j), GC)],
                              sem).start()

    def write_wait(buf, sem):
        pltpu.make_async_copy(buf, out_hbm.at[pl.ds(obase, GC)], sem).wait()

    def pipe_iter(j, cur, sem_gc, sem_wc, prv, sem_gp, sem_wp):
        @pl.when(j >= 2)
        def _():
            write_wait(cur, sem_wc)
        gather_start(j, cur, sem_gc)
        @pl.when(j >= 1)
        def _():
            gather_wait(prv, sem_gp)
            write_start(j - 1, prv, sem_wp)

    def pipe_body(j, _):
        @pl.when((j & 1) == 0)
        def _():
            pipe_iter(j, bufA, sem_gA, sem_wA, bufB, sem_gB, sem_wB)
        @pl.when((j & 1) == 1)
        def _():
            pipe_iter(j, bufB, sem_gB, sem_wB, bufA, sem_gA, sem_wA)
        return 0

    lax.fori_loop(0, nfull, pipe_body, 0)

    @pl.when(nfull >= 1)
    def _epilogue():
        jl = nfull - 1

        @pl.when((jl & 1) == 0)
        def _():
            gather_wait(bufA, sem_gA)
            write_start(jl, bufA, sem_wA)
            write_wait(bufA, sem_wA)
        @pl.when((jl & 1) == 1)
        def _():
            gather_wait(bufB, sem_gB)
            write_start(jl, bufB, sem_wB)
            write_wait(bufB, sem_wB)

        @pl.when(nfull >= 2)
        def _():
            @pl.when((jl & 1) == 0)
            def _():
                write_wait(bufB, sem_wB)
            @pl.when((jl & 1) == 1)
            def _():
                write_wait(bufA, sem_wA)

    @pl.when(has_bnd > 0)
    def _boundary():
        pltpu.async_copy(src_hbm.at[idx2.at[jbnd]], bufA, sem_gA).wait()

        def zrow(r, _):
            def zcol(k, _):
                bufA[r, pl.ds(k * 16, 16)] = jnp.zeros((16,), jnp.float32)
                return 0
            lax.fori_loop(0, D // 16, zcol, 0)
            return 0

        lax.fori_loop(remg, GC, zrow, 0)
        pltpu.sync_copy(bufA, out_hbm.at[pl.ds(ochunk(jbnd), GC)])

    cb = jnp.full((16,), count_b, jnp.int32)

    def mask_body(j, _):
        p = jnp.full((16,), p0 + j * 16, jnp.int32) + iota16
        maskbuf[pl.ds(j * 16, 16)] = jnp.where(p < cb, one16, z16)
        return 0

    lax.fori_loop(0, RPW // 16, mask_body, 0)
    pltpu.sync_copy(maskbuf, mask_hbm.at[pl.ds(obase + p0, RPW)])

    def zero_drain(j, _):
        pltpu.make_async_copy(zbuf, out_hbm.at[pl.ds(obase + p0, GC)],
                              sem_z).wait()
        return 0

    lax.fori_loop(jz0, NCHUNK, zero_drain, 0)


@functools.partial(
    pl.kernel,
    out_type=(jax.ShapeDtypeStruct((B * L, D), jnp.float32),
              jax.ShapeDtypeStruct((B * L,), jnp.int32)),
    mesh=plsc.VectorSubcoreMesh(core_axis_name="c", subcore_axis_name="s",
                                num_cores=NC, num_subcores=NS),
    scratch_types=[
        pltpu.VMEM((N,), jnp.int32),
        pltpu.VMEM((GC, D), jnp.float32),
        pltpu.VMEM((GC, D), jnp.float32),
        pltpu.VMEM((GC, D), jnp.float32),
        pltpu.VMEM((NCHUNK, GC), jnp.int32),
        pltpu.VMEM((RPW,), jnp.int32),
        pltpu.SemaphoreType.DMA,
        pltpu.SemaphoreType.DMA,
        pltpu.SemaphoreType.DMA,
        pltpu.SemaphoreType.DMA,
        pltpu.SemaphoreType.DMA,
    ],
    compiler_params=pltpu.CompilerParams(needs_layout_passes=False),
)
def _sc_kernel(src_hbm, batch_hbm, zeros_hbm, out_hbm, mask_hbm,
               batch_v, bufA, bufB, zbuf, idx2, maskbuf,
               sem_gA, sem_gB, sem_wA, sem_wB, sem_z):
    _sc_body(src_hbm, batch_hbm, zeros_hbm, out_hbm, mask_hbm,
             batch_v, bufA, bufB, zbuf, idx2, maskbuf,
             sem_gA, sem_gB, sem_wA, sem_wB, sem_z)


@jax.jit
def kernel(src, batch):
    zeros = jnp.zeros((GC, D), jnp.float32)
    padded_flat, mask_flat = _sc_kernel(src, batch.astype(jnp.int32), zeros)
    return padded_flat.reshape(B, L, D), mask_flat.reshape(B, L) != 0


# speedup vs baseline: 1.0694x; 1.0271x over previous
import functools

import jax
import jax.numpy as jnp
from jax import lax
from jax.experimental import pallas as pl
from jax.experimental.pallas import tpu as pltpu
from jax.experimental.pallas import tpu_sc as plsc

B = 16
L = 2048
D = 1024
N = 16384

NC = 2
NS = 16
NW = NC * NS
RPW = (B * L) // NW
GC = 32
NCHUNK = RPW // GC
ZP = 16
BH = N // 2


def _sc_body(src_hbm, batch_hbm, zeros_hbm, out_hbm, mask_hbm,
             batch_v, bufA, bufB, bufC, zbuf, idx2, maskbuf,
             sem_gA, sem_gB, sem_gC, sem_wA, sem_wB, sem_wC, sem_z):
    wid = lax.axis_index("s") * NC + lax.axis_index("c")
    b = wid & (B - 1)
    q = wid >> 4
    p0 = q * RPW

    pltpu.sync_copy(zeros_hbm, zbuf)

    z16 = jnp.zeros((16,), jnp.int32)
    one16 = jnp.full((16,), 1, jnp.int32)
    bvec = jnp.full((16,), b, jnp.int32)

    def scan_body(i, carry):
        lt, le = carry
        v = batch_v[pl.ds(i * 16, 16)]
        lt = lt + jnp.where(v < bvec, one16, z16)
        le = le + jnp.where(v <= bvec, one16, z16)
        return lt, le

    pltpu.sync_copy(batch_hbm.at[pl.ds(0, BH)], batch_v)
    lt, le = lax.fori_loop(0, BH // 16, scan_body, (z16, z16))
    pltpu.sync_copy(batch_hbm.at[pl.ds(BH, BH)], batch_v)
    lt, le = lax.fori_loop(0, BH // 16, scan_body, (lt, le))
    start_b = jnp.sum(lt)
    end_b = jnp.sum(le)
    count_b = end_b - start_b

    validb = jnp.minimum(count_b, L)
    src0 = start_b
    obase = b * L

    nfullg = validb // GC
    remg = validb - nfullg * GC

    nfull = (nfullg - q + 1) // 2
    has_bnd = jnp.where((remg > 0) & ((nfullg & 1) == q), 1, 0)
    jbnd = (nfullg - q) // 2

    iota16 = lax.iota(jnp.int32, 16)

    pmax = jnp.maximum(validb - 1, 0)
    qGC = q * GC

    def idx_body(k, _):
        j = k // (GC // 16)
        col = (k % (GC // 16)) * 16
        p = (jnp.full((16,), j * 2 * GC + qGC + col, jnp.int32) + iota16)
        pc = jnp.minimum(p, jnp.full((16,), pmax, jnp.int32))
        idx2[j, pl.ds(col, 16)] = pc + jnp.full((16,), src0, jnp.int32)
        return 0

    lax.fori_loop(0, RPW // 16, idx_body, 0)

    def ochunk(j):
        return obase + (2 * j + q) * GC

    jz0 = nfull + has_bnd

    def zero_start(k, _):
        row = ochunk(k // (GC // ZP)) + (k % (GC // ZP)) * ZP
        pltpu.make_async_copy(zbuf, out_hbm.at[pl.ds(row, ZP)],
                              sem_z).start()
        return 0

    lax.fori_loop(jz0 * (GC // ZP), NCHUNK * (GC // ZP), zero_start, 0)

    def gather_start(j, buf, sem):
        pltpu.make_async_copy(src_hbm.at[idx2.at[j]], buf, sem).start()

    def gather_wait(buf, sem):
        pltpu.make_async_copy(src_hbm.at[idx2.at[0]], buf, sem).wait()

    def write_start(j, buf, sem):
        pltpu.make_async_copy(buf, out_hbm.at[pl.ds(ochunk(j), GC)],
                              sem).start()

    def write_wait(buf, sem):
        pltpu.make_async_copy(buf, out_hbm.at[pl.ds(obase + p0, GC)],
                              sem).wait()

    bufs = (bufA, bufB, bufC)
    gsems = (sem_gA, sem_gB, sem_gC)
    wsems = (sem_wA, sem_wB, sem_wC)

    def pipe_iter(j, s):
        sp = (s + 2) % 3

        @pl.when(j >= 3)
        def _():
            write_wait(bufs[s], wsems[s])
        gather_start(j, bufs[s], gsems[s])

        @pl.when(j >= 1)
        def _():
            gather_wait(bufs[sp], gsems[sp])
            write_start(j - 1, bufs[sp], wsems[sp])

    def pipe_body(j, _):
        for s in range(3):
            @pl.when(j % 3 == s)
            def _(s=s):
                pipe_iter(j, s)
        return 0

    lax.fori_loop(0, nfull, pipe_body, 0)

    @pl.when(nfull >= 1)
    def _epilogue():
        jl = nfull - 1
        for s in range(3):
            @pl.when(jl % 3 == s)
            def _(s=s):
                gather_wait(bufs[s], gsems[s])
                write_start(jl, bufs[s], wsems[s])
                write_wait(bufs[s], wsems[s])

                @pl.when(nfull >= 2)
                def _():
                    write_wait(bufs[(s + 2) % 3], wsems[(s + 2) % 3])

                @pl.when(nfull >= 3)
                def _():
                    write_wait(bufs[(s + 1) % 3], wsems[(s + 1) % 3])

    @pl.when(has_bnd > 0)
    def _boundary():
        pltpu.async_copy(src_hbm.at[idx2.at[jbnd]], bufA, sem_gA).wait()

        def zrow(r, _):
            def zcol(k, _):
                bufA[r, pl.ds(k * 16, 16)] = jnp.zeros((16,), jnp.float32)
                return 0
            lax.fori_loop(0, D // 16, zcol, 0)
            return 0

        lax.fori_loop(remg, GC, zrow, 0)
        pltpu.sync_copy(bufA, out_hbm.at[pl.ds(ochunk(jbnd), GC)])

    cb = jnp.full((16,), count_b, jnp.int32)

    def mask_body(j, _):
        p = jnp.full((16,), p0 + j * 16, jnp.int32) + iota16
        maskbuf[pl.ds(j * 16, 16)] = jnp.where(p < cb, one16, z16)
        return 0

    lax.fori_loop(0, RPW // 16, mask_body, 0)
    pltpu.sync_copy(maskbuf, mask_hbm.at[pl.ds(obase + p0, RPW)])

    def zero_drain(k, _):
        pltpu.make_async_copy(zbuf, out_hbm.at[pl.ds(obase + p0, ZP)],
                              sem_z).wait()
        return 0

    lax.fori_loop(jz0 * (GC // ZP), NCHUNK * (GC // ZP), zero_drain, 0)


@functools.partial(
    pl.kernel,
    out_type=(jax.ShapeDtypeStruct((B * L, D), jnp.float32),
              jax.ShapeDtypeStruct((B * L,), jnp.int32)),
    mesh=plsc.VectorSubcoreMesh(core_axis_name="c", subcore_axis_name="s",
                                num_cores=NC, num_subcores=NS),
    scratch_types=[
        pltpu.VMEM((BH,), jnp.int32),
        pltpu.VMEM((GC, D), jnp.float32),
        pltpu.VMEM((GC, D), jnp.float32),
        pltpu.VMEM((GC, D), jnp.float32),
        pltpu.VMEM((ZP, D), jnp.float32),
        pltpu.VMEM((NCHUNK, GC), jnp.int32),
        pltpu.VMEM((RPW,), jnp.int32),
        pltpu.SemaphoreType.DMA,
        pltpu.SemaphoreType.DMA,
        pltpu.SemaphoreType.DMA,
        pltpu.SemaphoreType.DMA,
        pltpu.SemaphoreType.DMA,
        pltpu.SemaphoreType.DMA,
        pltpu.SemaphoreType.DMA,
    ],
    compiler_params=pltpu.CompilerParams(needs_layout_passes=False),
)
def _sc_kernel(src_hbm, batch_hbm, zeros_hbm, out_hbm, mask_hbm,
               batch_v, bufA, bufB, bufC, zbuf, idx2, maskbuf,
               sem_gA, sem_gB, sem_gC, sem_wA, sem_wB, sem_wC, sem_z):
    _sc_body(src_hbm, batch_hbm, zeros_hbm, out_hbm, mask_hbm,
             batch_v, bufA, bufB, bufC, zbuf, idx2, maskbuf,
             sem_gA, sem_gB, sem_gC, sem_wA, sem_wB, sem_wC, sem_z)


@jax.jit
def kernel(src, batch):
    zeros = jnp.zeros((ZP, D), jnp.float32)
    padded_flat, mask_flat = _sc_kernel(src, batch.astype(jnp.int32), zeros)
    return padded_flat.reshape(B, L, D), mask_flat.reshape(B, L) != 0


# speedup vs baseline: 1.0719x; 1.0023x over previous
import functools

import jax
import jax.numpy as jnp
from jax import lax
from jax.experimental import pallas as pl
from jax.experimental.pallas import tpu as pltpu
from jax.experimental.pallas import tpu_sc as plsc

B = 16
L = 2048
D = 1024
N = 16384

NC = 2
NS = 16
NW = NC * NS
RPW = (B * L) // NW
GC = 32
NCHUNK = RPW // GC
ZP = 16
BH = N // 2


def _sc_body(src_hbm, batch_hbm, zeros_hbm, out_hbm, mask_hbm,
             batch_v, bufA, bufB, bufC, zbuf, idx2, maskbuf,
             sem_gA, sem_gB, sem_gC, sem_wA, sem_wB, sem_wC, sem_z):
    wid = lax.axis_index("s") * NC + lax.axis_index("c")
    b = wid & (B - 1)
    q = wid >> 4
    p0 = q * RPW

    pltpu.sync_copy(zeros_hbm, zbuf)

    z16 = jnp.zeros((16,), jnp.int32)
    one16 = jnp.full((16,), 1, jnp.int32)
    bvec = jnp.full((16,), b, jnp.int32)

    def scan_body(i, carry):
        lt, le = carry
        v = batch_v[pl.ds(i * 16, 16)]
        lt = lt + jnp.where(v < bvec, one16, z16)
        le = le + jnp.where(v <= bvec, one16, z16)
        return lt, le

    pltpu.sync_copy(batch_hbm.at[pl.ds(0, BH)], batch_v)
    lt, le = lax.fori_loop(0, BH // 16, scan_body, (z16, z16))
    pltpu.sync_copy(batch_hbm.at[pl.ds(BH, BH)], batch_v)
    lt, le = lax.fori_loop(0, BH // 16, scan_body, (lt, le))
    start_b = jnp.sum(lt)
    end_b = jnp.sum(le)
    count_b = end_b - start_b

    validb = jnp.minimum(count_b, L)
    src0 = start_b
    obase = b * L

    nfullg = validb // GC
    remg = validb - nfullg * GC

    nfull = (nfullg - q + 1) // 2
    has_bnd = jnp.where((remg > 0) & ((nfullg & 1) == q), 1, 0)
    jbnd = (nfullg - q) // 2

    iota16 = lax.iota(jnp.int32, 16)

    def ochunk(j):
        return obase + (2 * j + q) * GC

    jz0 = nfull + has_bnd

    def zero_start(k, _):
        row = ochunk(k // (GC // ZP)) + (k % (GC // ZP)) * ZP
        pltpu.make_async_copy(zbuf, out_hbm.at[pl.ds(row, ZP)],
                              sem_z).start()
        return 0

    lax.fori_loop(jz0 * (GC // ZP), NCHUNK * (GC // ZP), zero_start, 0)

    pmax = jnp.maximum(validb - 1, 0)
    qGC = q * GC

    def idx_body(k, _):
        j = k // (GC // 16)
        col = (k % (GC // 16)) * 16
        p = (jnp.full((16,), j * 2 * GC + qGC + col, jnp.int32) + iota16)
        pc = jnp.minimum(p, jnp.full((16,), pmax, jnp.int32))
        idx2[j, pl.ds(col, 16)] = pc + jnp.full((16,), src0, jnp.int32)
        return 0

    lax.fori_loop(0, jz0 * (GC // 16), idx_body, 0)

    def gather_start(j, buf, sem):
        pltpu.make_async_copy(src_hbm.at[idx2.at[j]], buf, sem).start()

    def gather_wait(buf, sem):
        pltpu.make_async_copy(src_hbm.at[idx2.at[0]], buf, sem).wait()

    def write_start(j, buf, sem):
        pltpu.make_async_copy(buf, out_hbm.at[pl.ds(ochunk(j), GC)],
                              sem).start()

    def write_wait(buf, sem):
        pltpu.make_async_copy(buf, out_hbm.at[pl.ds(obase + p0, GC)],
                              sem).wait()

    bufs = (bufA, bufB, bufC)
    gsems = (sem_gA, sem_gB, sem_gC)
    wsems = (sem_wA, sem_wB, sem_wC)

    def pipe_iter(j, s):
        sp = (s + 2) % 3

        @pl.when(j >= 3)
        def _():
            write_wait(bufs[s], wsems[s])
        gather_start(j, bufs[s], gsems[s])

        @pl.when(j >= 1)
        def _():
            gather_wait(bufs[sp], gsems[sp])
            write_start(j - 1, bufs[sp], wsems[sp])

    def pipe_body(j, _):
        for s in range(3):
            @pl.when(j % 3 == s)
            def _(s=s):
                pipe_iter(j, s)
        return 0

    lax.fori_loop(0, nfull, pipe_body, 0)

    @pl.when(nfull >= 1)
    def _epilogue():
        jl = nfull - 1
        for s in range(3):
            @pl.when(jl % 3 == s)
            def _(s=s):
                gather_wait(bufs[s], gsems[s])
                write_start(jl, bufs[s], wsems[s])
                write_wait(bufs[s], wsems[s])

                @pl.when(nfull >= 2)
                def _():
                    write_wait(bufs[(s + 2) % 3], wsems[(s + 2) % 3])

                @pl.when(nfull >= 3)
                def _():
                    write_wait(bufs[(s + 1) % 3], wsems[(s + 1) % 3])

    @pl.when(has_bnd > 0)
    def _boundary():
        pltpu.async_copy(src_hbm.at[idx2.at[jbnd]], bufA, sem_gA).wait()

        def zrow(r, _):
            def zcol(k, _):
                bufA[r, pl.ds(k * 16, 16)] = jnp.zeros((16,), jnp.float32)
                return 0
            lax.fori_loop(0, D // 16, zcol, 0)
            return 0

        lax.fori_loop(remg, GC, zrow, 0)
        pltpu.sync_copy(bufA, out_hbm.at[pl.ds(ochunk(jbnd), GC)])

    cb = jnp.full((16,), count_b, jnp.int32)

    def mask_body(j, _):
        p = jnp.full((16,), p0 + j * 16, jnp.int32) + iota16
        maskbuf[pl.ds(j * 16, 16)] = jnp.where(p < cb, one16, z16)
        return 0

    lax.fori_loop(0, RPW // 16, mask_body, 0)
    pltpu.sync_copy(maskbuf, mask_hbm.at[pl.ds(obase + p0, RPW)])

    def zero_drain(k, _):
        pltpu.make_async_copy(zbuf, out_hbm.at[pl.ds(obase + p0, ZP)],
                              sem_z).wait()
        return 0

    lax.fori_loop(jz0 * (GC // ZP), NCHUNK * (GC // ZP), zero_drain, 0)


@functools.partial(
    pl.kernel,
    out_type=(jax.ShapeDtypeStruct((B * L, D), jnp.float32),
              jax.ShapeDtypeStruct((B * L,), jnp.int32)),
    mesh=plsc.VectorSubcoreMesh(core_axis_name="c", subcore_axis_name="s",
                                num_cores=NC, num_subcores=NS),
    scratch_types=[
        pltpu.VMEM((BH,), jnp.int32),
        pltpu.VMEM((GC, D), jnp.float32),
        pltpu.VMEM((GC, D), jnp.float32),
        pltpu.VMEM((GC, D), jnp.float32),
        pltpu.VMEM((ZP, D), jnp.float32),
        pltpu.VMEM((NCHUNK, GC), jnp.int32),
        pltpu.VMEM((RPW,), jnp.int32),
        pltpu.SemaphoreType.DMA,
        pltpu.SemaphoreType.DMA,
        pltpu.SemaphoreType.DMA,
        pltpu.SemaphoreType.DMA,
        pltpu.SemaphoreType.DMA,
        pltpu.SemaphoreType.DMA,
        pltpu.SemaphoreType.DMA,
    ],
    compiler_params=pltpu.CompilerParams(needs_layout_passes=False),
)
def _sc_kernel(src_hbm, batch_hbm, zeros_hbm, out_hbm, mask_hbm,
               batch_v, bufA, bufB, bufC, zbuf, idx2, maskbuf,
               sem_gA, sem_gB, sem_gC, sem_wA, sem_wB, sem_wC, sem_z):
    _sc_body(src_hbm, batch_hbm, zeros_hbm, out_hbm, mask_hbm,
             batch_v, bufA, bufB, bufC, zbuf, idx2, maskbuf,
             sem_gA, sem_gB, sem_gC, sem_wA, sem_wB, sem_wC, sem_z)


@jax.jit
def kernel(src, batch):
    zeros = jnp.zeros((ZP, D), jnp.float32)
    padded_flat, mask_flat = _sc_kernel(src, batch.astype(jnp.int32), zeros)
    return padded_flat.reshape(B, L, D), mask_flat.reshape(B, L) != 0
